# baseline (speedup 1.0000x reference)
import jax
import jax.numpy as jnp
from jax import lax
from jax.experimental import pallas as pl
from jax.experimental.pallas import tpu as pltpu
from jax.experimental.pallas import tpu_sc as plsc

D = 2048
E = 8
K = 2
R3 = 512
R2 = 512
B = 4096
EPS = 1e-5
SCALE = 10.0
TEMP = 0.5

TA = 1024
TC = 1024


def _stage_a(x_ref, nw_ref, u_ref, xs_ref):
    x = x_ref[...]
    var = jnp.mean(x * x, axis=-1, keepdims=True)
    xn = x * jax.lax.rsqrt(var + EPS) * nw_ref[...]
    xs_ref[...] = (jnp.tanh(
        jnp.dot(xn, u_ref[...], preferred_element_type=jnp.float32)
        * (1.0 / SCALE)) * SCALE).astype(jnp.bfloat16)


def _stage_c(xs_ref, w_ref, g_ref, v_ref, o_ref):
    xs = xs_ref[...]
    w = w_ref[...]
    acc = jnp.zeros((TC, R2), dtype=jnp.float32)
    for e in range(E):
        he = jnp.dot(xs, g_ref[e], preferred_element_type=jnp.float32)
        acc = acc + w[:, e:e + 1] * he
    o_ref[...] = jnp.dot(acc, v_ref[...], preferred_element_type=jnp.float32)


_SC_INFO = plsc.get_sparse_core_info()
_NC = _SC_INFO.num_cores
_NS = _SC_INFO.num_subcores
_NW = _NC * _NS
_TPW = B // _NW


def _lane_perm(v, idx2d):
    return lax.gather(
        v, idx2d,
        lax.GatherDimensionNumbers(offset_dims=(), collapsed_slice_dims=(0,),
                                   start_index_map=(0,)),
        (1,), mode=lax.GatherScatterMode.PROMISE_IN_BOUNDS)


def _router_sc(probs_hbm, w_hbm, pr_v, w_v):
    wid = lax.axis_index("s") * _NC + lax.axis_index("c")
    base = wid * _TPW * E
    pltpu.sync_copy(probs_hbm.at[pl.ds(base, _TPW * E)], pr_v)

    lane = lax.iota(jnp.int32, 16)
    eid = lane & (E - 1)

    def chunk(c, _):
        v = pr_v[pl.ds(c * 16, 16)]
        m1, i1 = v, eid
        m2 = jnp.full((16,), -jnp.inf, jnp.float32)
        i2 = jnp.full((16,), E, jnp.int32)
        for s in (1, 2, 4):
            pidx = (lane ^ s).reshape(16, 1)
            m1p = _lane_perm(m1, pidx)
            i1p = _lane_perm(i1, pidx)
            m2p = _lane_perm(m2, pidx)
            i2p = _lane_perm(i2, pidx)
            b1 = (m1p > m1) | ((m1p == m1) & (i1p < i1))
            new_m1 = jnp.where(b1, m1p, m1)
            new_i1 = jnp.where(b1, i1p, i1)
            ca = jnp.where(b1, m2p, m2)
            cia = jnp.where(b1, i2p, i2)
            cb = jnp.where(b1, m1, m1p)
            cib = jnp.where(b1, i1, i1p)
            b2 = (cb > ca) | ((cb == ca) & (cib < cia))
            m2 = jnp.where(b2, cb, ca)
            i2 = jnp.where(b2, cib, cia)
            m1, i1 = new_m1, new_i1
        s2 = m1 + m2
        w1 = m1 / s2
        w2 = m2 / s2
        w = jnp.where(eid == i1, w1, jnp.where(eid == i2, w2, 0.0))
        w_v[pl.ds(c * 16, 16)] = w
        return _

    lax.fori_loop(0, _TPW * E // 16, chunk, 0)
    pltpu.sync_copy(w_v, w_hbm.at[pl.ds(base, _TPW * E)])


@jax.jit
def kernel(x, norm_w, W_router, U, G, V):
    var = jnp.mean(x.astype(jnp.float32) ** 2, axis=-1, keepdims=True)
    xn = (x * jax.lax.rsqrt(var + EPS) * norm_w).astype(x.dtype)
    logits = xn @ W_router
    probs_full = jax.nn.softmax(logits / TEMP, axis=-1)

    mesh = plsc.VectorSubcoreMesh(core_axis_name="c", subcore_axis_name="s")
    w_flat = pl.kernel(
        _router_sc,
        mesh=mesh,
        out_type=jax.ShapeDtypeStruct((B * E,), jnp.float32),
        scratch_types=[
            pltpu.VMEM((_TPW * E,), jnp.float32),
            pltpu.VMEM((_TPW * E,), jnp.float32),
        ],
    )(probs_full.reshape(B * E))
    w = w_flat.reshape(B, E)

    xs = pl.pallas_call(
        _stage_a,
        grid=(B // TA,),
        in_specs=[
            pl.BlockSpec((TA, D), lambda i: (i, 0)),
            pl.BlockSpec((1, D), lambda i: (0, 0)),
            pl.BlockSpec((D, R3), lambda i: (0, 0)),
        ],
        out_specs=pl.BlockSpec((TA, R3), lambda i: (i, 0)),
        out_shape=jax.ShapeDtypeStruct((B, R3), jnp.bfloat16),
    )(x, norm_w.reshape(1, D), U)

    return pl.pallas_call(
        _stage_c,
        grid=(B // TC,),
        in_specs=[
            pl.BlockSpec((TC, R3), lambda i: (i, 0)),
            pl.BlockSpec((TC, E), lambda i: (i, 0)),
            pl.BlockSpec((E, R3, R2), lambda i: (0, 0, 0)),
            pl.BlockSpec((R2, D), lambda i: (0, 0)),
        ],
        out_specs=pl.BlockSpec((TC, D), lambda i: (i, 0)),
        out_shape=jax.ShapeDtypeStruct((B, D), jnp.float32),
    )(xs, w, G, V)
